# baseline (speedup 1.0000x reference)
import functools

import jax
import jax.numpy as jnp
from jax import lax
from jax.experimental import pallas as pl
from jax.experimental.pallas import tpu as pltpu
from jax.experimental.pallas import tpu_sc as plsc

NU = 50000
D = 64
H = 128
OUT = 64
B = 16384

NC = 2
NS = 16
NW = NC * NS
G = 128
W = 32
CW = 16
RT = B // NS
RW = B // NW
IDX_ROWS = B // G
_NSUB = 4
_SUBR = RT // _NSUB

_SC_MESH = plsc.VectorSubcoreMesh(core_axis_name="c", subcore_axis_name="s")
_SC_PARAMS = pltpu.CompilerParams(use_tc_tiling_on_sc=False)


def _dot(a, b, dims=(((1,), (0,)), ((), ()))):
    return jax.lax.dot_general(
        a.astype(jnp.bfloat16), b.astype(jnp.bfloat16), dims,
        preferred_element_type=jnp.float32)


def _dot32(a, b):
    return jax.lax.dot_general(
        a, b, (((1,), (0,)), ((), ())),
        precision=jax.lax.Precision.HIGHEST,
        preferred_element_type=jnp.float32)


def _inv_counts(cv):
    vr = cv.shape[0]
    rb = vr * (G // CW)
    inv = 1.0 / jnp.maximum(cv, 1.0)
    rowsel = (jax.lax.broadcasted_iota(jnp.int32, (rb, vr), 0) // 8
              == jax.lax.broadcasted_iota(jnp.int32, (rb, vr), 1)
              ).astype(jnp.float32)
    e = _dot32(rowsel, inv)
    mask = (jax.lax.broadcasted_iota(jnp.int32, (rb, G), 1) // CW
            == jax.lax.broadcasted_iota(jnp.int32, (rb, G), 0) % 8
            ).astype(jnp.float32)
    return _dot32(e * mask, jnp.ones((G, G), jnp.float32)) * (1.0 / CW)


def _fill(ref, value):
    r, c = ref.shape

    def body(i, carry):
        for j in range(c // 16):
            ref[i, pl.ds(j * 16, 16)] = jnp.full((16,), value, ref.dtype)
        return carry

    lax.fori_loop(0, r, body, 0)


@functools.partial(
    pl.kernel,
    out_type=(
        jax.ShapeDtypeStruct((2 * B, H), jnp.float32),
        jax.ShapeDtypeStruct((2 * B, CW), jnp.float32),
    ),
    mesh=_SC_MESH,
    scratch_types=[
        pltpu.VMEM((8, G), jnp.int32),
        pltpu.VMEM((8, G), jnp.int32),
        pltpu.VMEM((G, CW), jnp.float32),
        pltpu.VMEM((G, CW), jnp.float32),
        pltpu.VMEM((_SUBR, H), jnp.float32),
        pltpu.VMEM((_SUBR, H), jnp.float32),
        pltpu.VMEM((_SUBR, CW), jnp.float32),
        pltpu.VMEM_SHARED((NU, CW), jnp.float32),
    ] + [pltpu.SemaphoreType.DMA] * 6,
    compiler_params=_SC_PARAMS,
)
def _k1(ids_hbm, xtu_hbm, xti_hbm, x_hbm, cnt_hbm,
        idxg_v, idxc_v, zeros_v, ones_v, buf0, buf1, cbuf_v, acc_sh,
        sp0, sp1, sa0, sa1, ss0, ss1):
    c = lax.axis_index("c")
    s = lax.axis_index("s")
    wid = c * NS + s
    bufs = (buf0, buf1)
    sp = (sp0, sp1)
    ss = (ss0, ss1)

    dg = pltpu.async_copy(ids_hbm.at[pl.ds(4 * wid, 4)],
                          idxg_v.at[pl.ds(0, 4)], sp0)
    dg2 = pltpu.async_copy(ids_hbm.at[pl.ds(IDX_ROWS + 4 * wid, 4)],
                           idxg_v.at[pl.ds(4, 4)], sp0)
    dc = pltpu.async_copy(ids_hbm.at[pl.ds(c * IDX_ROWS + 8 * s, 8)],
                          idxc_v, sp1)
    _fill(zeros_v, 0.0)
    _fill(ones_v, 1.0)
    dg.wait()
    dg2.wait()
    dc.wait()

    zs = [pltpu.async_copy(zeros_v, acc_sh.at[idxc_v.at[j]], sa0)
          for j in range(8)]
    for d in zs:
        d.wait()
    plsc.subcore_barrier()
    ad = [pltpu.async_copy(ones_v, acc_sh.at[idxc_v.at[j]], sa1, add=True)
          for j in range(8)]

    subs = [(xtu_hbm, 0, 0), (xtu_hbm, 2, _SUBR),
            (xti_hbm, 0, B), (xti_hbm, 2, B + _SUBR)]
    st = [None, None]
    for p, (tab, jbase, obase) in enumerate(subs):
        if st[p % 2] is not None:
            st[p % 2].wait()
        gs = [pltpu.async_copy(
                  tab.at[idxg_v.at[(4 if p >= 2 else 0) + jbase + q]],
                  bufs[p % 2].at[pl.ds(q * G, G)], sp[p % 2])
              for q in range(2)]
        for d in gs:
            d.wait()
        st[p % 2] = pltpu.async_copy(
            bufs[p % 2], x_hbm.at[pl.ds(obase + RW * wid, _SUBR)], ss[p % 2])
    for d in st:
        d.wait()

    for d in ad:
        d.wait()
    plsc.subcore_barrier()
    cst = []
    for p in range(_NSUB):
        gs = [pltpu.async_copy(acc_sh.at[idxc_v.at[2 * p + q]],
                               cbuf_v.at[pl.ds(q * G, G)], sp0)
              for q in range(2)]
        for d in gs:
            d.wait()
        cst.append(pltpu.async_copy(
            cbuf_v, cnt_hbm.at[pl.ds(c * B + RT * s + p * _SUBR, _SUBR)],
            ss0))
        if cst and p < _NSUB - 1:
            cst[-1].wait()
    for d in cst[-1:]:
        d.wait()


@functools.partial(
    pl.kernel,
    out_type=jax.ShapeDtypeStruct((2 * B, H), jnp.float32),
    mesh=_SC_MESH,
    scratch_types=[
        pltpu.VMEM((8, G), jnp.int32),
        pltpu.VMEM((G, W), jnp.float32),
        pltpu.VMEM((_SUBR, W), jnp.float32),
        pltpu.VMEM((_SUBR, W), jnp.float32),
        pltpu.VMEM_SHARED((NU, W), jnp.float32),
    ] + [pltpu.SemaphoreType.DMA] * 6,
    compiler_params=_SC_PARAMS,
)
def _kagg(ids_hbm, x_hbm, raw_hbm,
          idx_v, zeros_v, buf0, buf1, acc_sh, sp0, sp1, sa0, sa1, ss0, ss1):
    bufs = (buf0, buf1)
    sp = (sp0, sp1)
    sa = (sa0, sa1)
    ss = (ss0, ss1)

    c = lax.axis_index("c")
    s = lax.axis_index("s")
    src_base = (1 - c) * B + RT * s
    dst_base = c * B + RT * s

    dl = pltpu.async_copy(ids_hbm.at[pl.ds(c * IDX_ROWS + 8 * s, 8)],
                          idx_v, sp0)
    _fill(zeros_v, 0.0)
    dl.wait()

    def zero_phase():
        ds = [pltpu.async_copy(zeros_v, acc_sh.at[idx_v.at[j]], sa0)
              for j in range(8)]
        for d in ds:
            d.wait()

    def add_phase(h):
        def load(p):
            return pltpu.async_copy(
                x_hbm.at[pl.ds(src_base + p * _SUBR, _SUBR),
                         pl.ds(W * h, W)],
                bufs[p % 2], sp[p % 2])

        loads = [load(0), load(1), None, None]
        pend = {}
        for p in range(_NSUB):
            loads[p].wait()
            pend[p] = [
                pltpu.async_copy(bufs[p % 2].at[pl.ds(q * G, G)],
                                 acc_sh.at[idx_v.at[2 * p + q]],
                                 sa[p % 2], add=True)
                for q in range(2)]
            if p + 2 < _NSUB:
                for d in pend[p]:
                    d.wait()
                del pend[p]
                loads[p + 2] = load(p + 2)
        for ds in pend.values():
            for d in ds:
                d.wait()

    def gather_phase(h):
        st = [None, None]
        for p in range(_NSUB):
            if st[p % 2] is not None:
                st[p % 2].wait()
            gs = [pltpu.async_copy(acc_sh.at[idx_v.at[2 * p + q]],
                                   bufs[p % 2].at[pl.ds(q * G, G)],
                                   sp[p % 2])
                  for q in range(2)]
            for d in gs:
                d.wait()
            st[p % 2] = pltpu.async_copy(
                bufs[p % 2],
                raw_hbm.at[pl.ds(dst_base + p * _SUBR, _SUBR),
                           pl.ds(W * h, W)],
                ss[p % 2])
        for d in st:
            d.wait()

    for h in range(H // W):
        zero_phase()
        plsc.subcore_barrier()
        add_phase(h)
        plsc.subcore_barrier()
        gather_phase(h)
        if h < H // W - 1:
            plsc.subcore_barrier()


_RB = 1024
_GRID2 = 2 * B // _RB
_GRID1 = B // _RB
_TB = 4096


def _row_spec(w, offset_blocks=0):
    return pl.BlockSpec((_RB, w), lambda i: (i + offset_blocks, 0))


def _full_spec(r, w):
    return pl.BlockSpec((r, w), lambda i: (0, 0))


def _tabproj_body(utT, itT, winT, b, xu, xi):
    xu[...] = jax.nn.relu(
        _dot(utT[...], winT[...], (((0,), (0,)), ((), ()))) + b[...])
    xi[...] = jax.nn.relu(
        _dot(itT[...], winT[...], (((0,), (0,)), ((), ()))) + b[...])


def _tabproj(utT, itT, winT, b):
    return pl.pallas_call(
        _tabproj_body,
        grid=(pl.cdiv(NU, _TB),),
        in_specs=[pl.BlockSpec((D, _TB), lambda i: (0, i)),
                  pl.BlockSpec((D, _TB), lambda i: (0, i)),
                  _full_spec(D, H), _full_spec(1, H)],
        out_specs=[pl.BlockSpec((_TB, H), lambda i: (i, 0)),
                   pl.BlockSpec((_TB, H), lambda i: (i, 0))],
        out_shape=[jax.ShapeDtypeStruct((NU, H), jnp.float32)] * 2,
    )(utT, itT, winT, b)


def _layer_body(raw, x, cnt, wlT, wrT, blr, sg, bb, x2):
    ic = _inv_counts(cnt[...])
    y = _dot(raw[...] * ic, wlT[...]) + _dot(x[...], wrT[...]) + blr[...]
    x2[...] = jax.nn.relu(y * sg[...] + bb[...])


def _layer(raw, x, cnt, wlT, wrT, blr, sg, bb):
    return pl.pallas_call(
        _layer_body,
        grid=(_GRID2,),
        in_specs=[_row_spec(H), _row_spec(H),
                  pl.BlockSpec((_RB * CW // G, G), lambda i: (i, 0))]
        + [_full_spec(H, H)] * 2 + [_full_spec(1, H)] * 3,
        out_specs=_row_spec(H),
        out_shape=jax.ShapeDtypeStruct((2 * B, H), jnp.float32),
    )(raw, x, cnt, wlT, wrT, blr, sg, bb)


def _head_body(rawu, rawi, xu, xi, cntu, cnti, wlT, wrT, blr, sg, bb,
               wout, bo, pw1uT, pw1iT, pb1, pw2, pb2,
               predT, yuT_o, yiT_o):
    icu = _inv_counts(cntu[...])
    ici = _inv_counts(cnti[...])
    tu = _dot(rawu[...] * icu, wlT[...]) + _dot(xu[...], wrT[...]) + blr[...]
    ti = _dot(rawi[...] * ici, wlT[...]) + _dot(xi[...], wrT[...]) + blr[...]
    xu3 = jax.nn.relu(tu * sg[...] + bb[...])
    xi3 = jax.nn.relu(ti * sg[...] + bb[...])
    yuT = _dot(wout[...], xu3, (((1,), (1,)), ((), ()))) + bo[...]
    yiT = _dot(wout[...], xi3, (((1,), (1,)), ((), ()))) + bo[...]
    h = jax.nn.relu(_dot(yuT, pw1uT[...], (((0,), (0,)), ((), ())))
                    + _dot(yiT, pw1iT[...], (((0,), (0,)), ((), ())))
                    + pb1[...])
    zT = _dot(pw2[...], h, (((1,), (1,)), ((), ()))) + pb2[...]
    predT[...] = jax.nn.sigmoid(zT)
    yuT_o[...] = yuT
    yiT_o[...] = yiT


def _head(raw, x, cnt, wlT, wrT, blr, sg, bb,
          wout, bo, pw1uT, pw1iT, pb1, pw2, pb2):
    nb = _GRID1
    return pl.pallas_call(
        _head_body,
        grid=(nb,),
        in_specs=[_row_spec(H), _row_spec(H, nb),
                  _row_spec(H), _row_spec(H, nb),
                  pl.BlockSpec((_RB * CW // G, G), lambda i: (i, 0)),
                  pl.BlockSpec((_RB * CW // G, G), lambda i: (i + nb, 0))]
        + [_full_spec(H, H)] * 2 + [_full_spec(1, H)] * 3
        + [_full_spec(OUT, H), pl.BlockSpec((OUT, 1), lambda i: (0, 0))]
        + [_full_spec(OUT, H)] * 2 + [_full_spec(1, H)] * 2
        + [_full_spec(1, 1)],
        out_specs=[pl.BlockSpec((1, _RB), lambda i: (0, i)),
                   pl.BlockSpec((OUT, _RB), lambda i: (0, i)),
                   pl.BlockSpec((OUT, _RB), lambda i: (0, i))],
        out_shape=[jax.ShapeDtypeStruct((1, B), jnp.float32),
                   jax.ShapeDtypeStruct((OUT, B), jnp.float32),
                   jax.ShapeDtypeStruct((OUT, B), jnp.float32)],
    )(raw, raw, x, x, cnt, cnt, wlT, wrT, blr, sg, bb,
      wout, bo, pw1uT, pw1iT, pb1, pw2, pb2)


def kernel(user_ids, item_ids, user_table, item_table, Win, bin_,
           l1_Wl, l1_bl, l1_Wr, l1_br, bn1_g, bn1_b,
           l2_Wl, l2_bl, l2_Wr, l2_br, bn2_g, bn2_b,
           Wout, bout, pW1, pb1, pW2, pb2):
    sc = 1.0 / jnp.sqrt(jnp.float32(1.0 + 1e-5))
    ids2 = jnp.concatenate([user_ids.reshape(IDX_ROWS, G),
                            item_ids.reshape(IDX_ROWS, G)], axis=0)

    xtu, xti = _tabproj(user_table.T, item_table.T, Win.T,
                        bin_.reshape(1, H))
    x, cnt = _k1(ids2, xtu, xti)
    cntf = cnt.reshape(2 * B * CW // G, G)

    raw = _kagg(ids2, x)
    x = _layer(raw, x, cntf,
               l1_Wl.T, l1_Wr.T, (l1_bl + l1_br).reshape(1, H),
               (sc * bn1_g).reshape(1, H), bn1_b.reshape(1, H))

    raw = _kagg(ids2, x)
    predT, yuT, yiT = _head(raw, x, cntf,
                            l2_Wl.T, l2_Wr.T, (l2_bl + l2_br).reshape(1, H),
                            (sc * bn2_g).reshape(1, H), bn2_b.reshape(1, H),
                            Wout, bout.reshape(OUT, 1),
                            pW1[:, :OUT].T, pW1[:, OUT:].T,
                            pb1.reshape(1, H),
                            pW2.reshape(1, H), pb2.reshape(1, 1))

    return (predT.reshape(B), yuT.T, yiT.T)


# speedup vs baseline: 1.2636x; 1.2636x over previous
import functools

import jax
import jax.numpy as jnp
from jax import lax
from jax.experimental import pallas as pl
from jax.experimental.pallas import tpu as pltpu
from jax.experimental.pallas import tpu_sc as plsc

NU = 50000
D = 64
H = 128
OUT = 64
B = 16384

NC = 2
NS = 16
NW = NC * NS
G = 128
W = 32
CW = 16
RT = B // NS
RW = B // NW
IDX_ROWS = B // G
_NSUB = 4
_SUBR = RT // _NSUB

_SC_MESH = plsc.VectorSubcoreMesh(core_axis_name="c", subcore_axis_name="s")
_SC_PARAMS = pltpu.CompilerParams(use_tc_tiling_on_sc=False)


def _dot(a, b, dims=(((1,), (0,)), ((), ()))):
    return jax.lax.dot_general(
        a.astype(jnp.bfloat16), b.astype(jnp.bfloat16), dims,
        preferred_element_type=jnp.float32)


def _dot32(a, b):
    return jax.lax.dot_general(
        a, b, (((1,), (0,)), ((), ())),
        precision=jax.lax.Precision.HIGHEST,
        preferred_element_type=jnp.float32)


def _inv_counts(cv):
    vr = cv.shape[0]
    rb = vr * (G // CW)
    inv = 1.0 / jnp.maximum(cv, 1.0)
    rowsel = (jax.lax.broadcasted_iota(jnp.int32, (rb, vr), 0) // 8
              == jax.lax.broadcasted_iota(jnp.int32, (rb, vr), 1)
              ).astype(jnp.float32)
    e = _dot32(rowsel, inv)
    mask = (jax.lax.broadcasted_iota(jnp.int32, (rb, G), 1) // CW
            == jax.lax.broadcasted_iota(jnp.int32, (rb, G), 0) % 8
            ).astype(jnp.float32)
    return _dot32(e * mask, jnp.ones((G, G), jnp.float32)) * (1.0 / CW)


def _fill(ref, value):
    r, c = ref.shape

    def body(i, carry):
        for j in range(c // 16):
            ref[i, pl.ds(j * 16, 16)] = jnp.full((16,), value, ref.dtype)
        return carry

    lax.fori_loop(0, r, body, 0)


@functools.partial(
    pl.kernel,
    out_type=(
        jax.ShapeDtypeStruct((2 * B, H), jnp.float32),
        jax.ShapeDtypeStruct((2 * B, CW), jnp.float32),
    ),
    mesh=_SC_MESH,
    scratch_types=[
        pltpu.VMEM((8, G), jnp.int32),
        pltpu.VMEM((8, G), jnp.int32),
        pltpu.VMEM((G, CW), jnp.float32),
        pltpu.VMEM((G, CW), jnp.float32),
        pltpu.VMEM((_SUBR, H), jnp.float32),
        pltpu.VMEM((_SUBR, H), jnp.float32),
        pltpu.VMEM((_SUBR, CW), jnp.float32),
        pltpu.VMEM_SHARED((NU, CW), jnp.float32),
    ] + [pltpu.SemaphoreType.DMA] * 6,
    compiler_params=_SC_PARAMS,
)
def _k1(ids_hbm, xtu_hbm, xti_hbm, x_hbm, cnt_hbm,
        idxg_v, idxc_v, zeros_v, ones_v, buf0, buf1, cbuf_v, acc_sh,
        sp0, sp1, sa0, sa1, ss0, ss1):
    c = lax.axis_index("c")
    s = lax.axis_index("s")
    wid = c * NS + s
    bufs = (buf0, buf1)
    sp = (sp0, sp1)
    ss = (ss0, ss1)

    dg = pltpu.async_copy(ids_hbm.at[pl.ds(4 * wid, 4)],
                          idxg_v.at[pl.ds(0, 4)], sp0)
    dg2 = pltpu.async_copy(ids_hbm.at[pl.ds(IDX_ROWS + 4 * wid, 4)],
                           idxg_v.at[pl.ds(4, 4)], sp0)
    dc = pltpu.async_copy(ids_hbm.at[pl.ds(c * IDX_ROWS + 8 * s, 8)],
                          idxc_v, sp1)
    _fill(zeros_v, 0.0)
    _fill(ones_v, 1.0)
    dg.wait()
    dg2.wait()
    dc.wait()

    zs = [pltpu.async_copy(zeros_v, acc_sh.at[idxc_v.at[j]], sa0)
          for j in range(8)]
    for d in zs:
        d.wait()
    plsc.subcore_barrier()
    ad = [pltpu.async_copy(ones_v, acc_sh.at[idxc_v.at[j]], sa1, add=True)
          for j in range(8)]

    subs = [(xtu_hbm, 0, 0), (xtu_hbm, 2, _SUBR),
            (xti_hbm, 0, B), (xti_hbm, 2, B + _SUBR)]
    st = [None, None]
    for p, (tab, jbase, obase) in enumerate(subs):
        if st[p % 2] is not None:
            st[p % 2].wait()
        gs = [pltpu.async_copy(
                  tab.at[idxg_v.at[(4 if p >= 2 else 0) + jbase + q]],
                  bufs[p % 2].at[pl.ds(q * G, G)], sp[p % 2])
              for q in range(2)]
        for d in gs:
            d.wait()
        st[p % 2] = pltpu.async_copy(
            bufs[p % 2], x_hbm.at[pl.ds(obase + RW * wid, _SUBR)], ss[p % 2])
    for d in st:
        d.wait()

    for d in ad:
        d.wait()
    plsc.subcore_barrier()
    cst = []
    for p in range(_NSUB):
        gs = [pltpu.async_copy(acc_sh.at[idxc_v.at[2 * p + q]],
                               cbuf_v.at[pl.ds(q * G, G)], sp0)
              for q in range(2)]
        for d in gs:
            d.wait()
        cst.append(pltpu.async_copy(
            cbuf_v, cnt_hbm.at[pl.ds(c * B + RT * s + p * _SUBR, _SUBR)],
            ss0))
        if cst and p < _NSUB - 1:
            cst[-1].wait()
    for d in cst[-1:]:
        d.wait()


@functools.partial(
    pl.kernel,
    out_type=jax.ShapeDtypeStruct((2 * B, H), jnp.float32),
    mesh=_SC_MESH,
    scratch_types=[
        pltpu.VMEM((8, G), jnp.int32),
        pltpu.VMEM((G, W), jnp.float32),
        pltpu.VMEM((_SUBR, W), jnp.float32),
        pltpu.VMEM((_SUBR, W), jnp.float32),
        pltpu.VMEM_SHARED((NU, W), jnp.float32),
    ] + [pltpu.SemaphoreType.DMA] * 6,
    compiler_params=_SC_PARAMS,
)
def _kagg(ids_hbm, x_hbm, raw_hbm,
          idx_v, zeros_v, buf0, buf1, acc_sh, sp0, sp1, sa0, sa1, ss0, ss1):
    bufs = (buf0, buf1)
    sp = (sp0, sp1)
    sa = (sa0, sa1)
    ss = (ss0, ss1)

    c = lax.axis_index("c")
    s = lax.axis_index("s")
    src_base = (1 - c) * B + RT * s
    dst_base = c * B + RT * s

    dl = pltpu.async_copy(ids_hbm.at[pl.ds(c * IDX_ROWS + 8 * s, 8)],
                          idx_v, sp0)
    _fill(zeros_v, 0.0)
    dl.wait()

    def zero_phase():
        ds = [pltpu.async_copy(zeros_v, acc_sh.at[idx_v.at[j]], sa0)
              for j in range(8)]
        for d in ds:
            d.wait()

    def add_phase(h):
        def load(p):
            return pltpu.async_copy(
                x_hbm.at[pl.ds(src_base + p * _SUBR, _SUBR),
                         pl.ds(W * h, W)],
                bufs[p % 2], sp[p % 2])

        loads = [load(0), load(1), None, None]
        pend = {}
        for p in range(_NSUB):
            loads[p].wait()
            pend[p] = [
                pltpu.async_copy(bufs[p % 2].at[pl.ds(q * G, G)],
                                 acc_sh.at[idx_v.at[2 * p + q]],
                                 sa[p % 2], add=True)
                for q in range(2)]
            if p + 2 < _NSUB:
                for d in pend[p]:
                    d.wait()
                del pend[p]
                loads[p + 2] = load(p + 2)
        for ds in pend.values():
            for d in ds:
                d.wait()

    def gather_phase(h):
        st = [None, None]
        for p in range(_NSUB):
            if st[p % 2] is not None:
                st[p % 2].wait()
            gs = [pltpu.async_copy(acc_sh.at[idx_v.at[2 * p + q]],
                                   bufs[p % 2].at[pl.ds(q * G, G)],
                                   sp[p % 2])
                  for q in range(2)]
            for d in gs:
                d.wait()
            st[p % 2] = pltpu.async_copy(
                bufs[p % 2],
                raw_hbm.at[pl.ds(dst_base + p * _SUBR, _SUBR),
                           pl.ds(W * h, W)],
                ss[p % 2])
        for d in st:
            d.wait()

    for h in range(H // W):
        zero_phase()
        plsc.subcore_barrier()
        add_phase(h)
        plsc.subcore_barrier()
        gather_phase(h)
        if h < H // W - 1:
            plsc.subcore_barrier()


_RB = 1024
_GRID2 = 2 * B // _RB
_GRID1 = B // _RB
_TB = 4096


def _row_spec(w, offset_blocks=0):
    return pl.BlockSpec((_RB, w), lambda i: (i + offset_blocks, 0))


def _full_spec(r, w):
    return pl.BlockSpec((r, w), lambda i: (0, 0))


def _tabproj_body(utT, itT, winT, b, xu, xi):
    xu[...] = jax.nn.relu(
        _dot(utT[...], winT[...], (((0,), (0,)), ((), ()))) + b[...])
    xi[...] = jax.nn.relu(
        _dot(itT[...], winT[...], (((0,), (0,)), ((), ()))) + b[...])


def _tabproj(utT, itT, winT, b):
    return pl.pallas_call(
        _tabproj_body,
        grid=(pl.cdiv(NU, _TB),),
        in_specs=[pl.BlockSpec((D, _TB), lambda i: (0, i)),
                  pl.BlockSpec((D, _TB), lambda i: (0, i)),
                  _full_spec(D, H), _full_spec(1, H)],
        out_specs=[pl.BlockSpec((_TB, H), lambda i: (i, 0)),
                   pl.BlockSpec((_TB, H), lambda i: (i, 0))],
        out_shape=[jax.ShapeDtypeStruct((NU, H), jnp.float32)] * 2,
    )(utT, itT, winT, b)


def _layer_body(raw, x, cnt, wlT, wrT, blr, sg, bb, x2):
    ic = 1.0 / jnp.maximum(cnt[...][:, :1], 1.0)
    y = _dot(raw[...] * ic, wlT[...]) + _dot(x[...], wrT[...]) + blr[...]
    x2[...] = jax.nn.relu(y * sg[...] + bb[...])


def _layer(raw, x, cnt, wlT, wrT, blr, sg, bb):
    return pl.pallas_call(
        _layer_body,
        grid=(_GRID2,),
        in_specs=[_row_spec(H), _row_spec(H), _row_spec(CW)]
        + [_full_spec(H, H)] * 2 + [_full_spec(1, H)] * 3,
        out_specs=_row_spec(H),
        out_shape=jax.ShapeDtypeStruct((2 * B, H), jnp.float32),
    )(raw, x, cnt, wlT, wrT, blr, sg, bb)


def _head_body(rawu, rawi, xu, xi, cntu, cnti, wlT, wrT, blr, sg, bb,
               wout, bo, pw1uT, pw1iT, pb1, pw2, pb2,
               predT, yuT_o, yiT_o):
    icu = 1.0 / jnp.maximum(cntu[...][:, :1], 1.0)
    ici = 1.0 / jnp.maximum(cnti[...][:, :1], 1.0)
    tu = _dot(rawu[...] * icu, wlT[...]) + _dot(xu[...], wrT[...]) + blr[...]
    ti = _dot(rawi[...] * ici, wlT[...]) + _dot(xi[...], wrT[...]) + blr[...]
    xu3 = jax.nn.relu(tu * sg[...] + bb[...])
    xi3 = jax.nn.relu(ti * sg[...] + bb[...])
    yuT = _dot(wout[...], xu3, (((1,), (1,)), ((), ()))) + bo[...]
    yiT = _dot(wout[...], xi3, (((1,), (1,)), ((), ()))) + bo[...]
    h = jax.nn.relu(_dot(yuT, pw1uT[...], (((0,), (0,)), ((), ())))
                    + _dot(yiT, pw1iT[...], (((0,), (0,)), ((), ())))
                    + pb1[...])
    zT = _dot(pw2[...], h, (((1,), (1,)), ((), ()))) + pb2[...]
    predT[...] = jax.nn.sigmoid(zT)
    yuT_o[...] = yuT
    yiT_o[...] = yiT


def _head(raw, x, cnt, wlT, wrT, blr, sg, bb,
          wout, bo, pw1uT, pw1iT, pb1, pw2, pb2):
    nb = _GRID1
    return pl.pallas_call(
        _head_body,
        grid=(nb,),
        in_specs=[_row_spec(H), _row_spec(H, nb),
                  _row_spec(H), _row_spec(H, nb),
                  _row_spec(CW), _row_spec(CW, nb)]
        + [_full_spec(H, H)] * 2 + [_full_spec(1, H)] * 3
        + [_full_spec(OUT, H), pl.BlockSpec((OUT, 1), lambda i: (0, 0))]
        + [_full_spec(OUT, H)] * 2 + [_full_spec(1, H)] * 2
        + [_full_spec(1, 1)],
        out_specs=[pl.BlockSpec((1, _RB), lambda i: (0, i)),
                   pl.BlockSpec((OUT, _RB), lambda i: (0, i)),
                   pl.BlockSpec((OUT, _RB), lambda i: (0, i))],
        out_shape=[jax.ShapeDtypeStruct((1, B), jnp.float32),
                   jax.ShapeDtypeStruct((OUT, B), jnp.float32),
                   jax.ShapeDtypeStruct((OUT, B), jnp.float32)],
    )(raw, raw, x, x, cnt, cnt, wlT, wrT, blr, sg, bb,
      wout, bo, pw1uT, pw1iT, pb1, pw2, pb2)


def kernel(user_ids, item_ids, user_table, item_table, Win, bin_,
           l1_Wl, l1_bl, l1_Wr, l1_br, bn1_g, bn1_b,
           l2_Wl, l2_bl, l2_Wr, l2_br, bn2_g, bn2_b,
           Wout, bout, pW1, pb1, pW2, pb2):
    sc = 1.0 / jnp.sqrt(jnp.float32(1.0 + 1e-5))
    ids2 = jnp.concatenate([user_ids.reshape(IDX_ROWS, G),
                            item_ids.reshape(IDX_ROWS, G)], axis=0)

    xtu, xti = _tabproj(user_table.T, item_table.T, Win.T,
                        bin_.reshape(1, H))
    x, cnt = _k1(ids2, xtu, xti)

    raw = _kagg(ids2, x)
    x = _layer(raw, x, cnt,
               l1_Wl.T, l1_Wr.T, (l1_bl + l1_br).reshape(1, H),
               (sc * bn1_g).reshape(1, H), bn1_b.reshape(1, H))

    raw = _kagg(ids2, x)
    predT, yuT, yiT = _head(raw, x, cnt,
                            l2_Wl.T, l2_Wr.T, (l2_bl + l2_br).reshape(1, H),
                            (sc * bn2_g).reshape(1, H), bn2_b.reshape(1, H),
                            Wout, bout.reshape(OUT, 1),
                            pW1[:, :OUT].T, pW1[:, OUT:].T,
                            pb1.reshape(1, H),
                            pW2.reshape(1, H), pb2.reshape(1, 1))

    return (predT.reshape(B), yuT.T, yiT.T)
